# baseline (speedup 1.0000x reference)
import jax
import jax.numpy as jnp
from jax import lax
from jax.experimental import pallas as pl
from jax.experimental.pallas import tpu as pltpu, tpu_sc as plsc

_D = 768
_B = 16
_NSLOT = 4
_NC, _NS = 2, 16
_NW = _NC * _NS
_NP = _D // 32
_SKEW = 4


def _sc_body(x_hbm, idx_hbm, pe_hbm, out_hbm, dump_hbm, pe_v, *rest):
    xbs = rest[0:4]
    ibs = rest[4:8]
    lxs = rest[8:12]
    lis = rest[12:16]
    sts = rest[16:20]

    c = lax.axis_index("c")
    s = lax.axis_index("s")
    wid = s * _NC + c
    rows_per_w = x_hbm.shape[0] // _NW
    chunks = rows_per_w // _B
    base0 = wid * rows_per_w

    pltpu.sync_copy(pe_hbm, pe_v)

    def start_load(g, t):
        b = base0 + jnp.minimum(g, chunks - 1) * _B
        pltpu.async_copy(x_hbm.at[pl.ds(b, _B)], xbs[t], lxs[t])
        pltpu.async_copy(idx_hbm.at[pl.ds(b, _B)], ibs[t], lis[t])

    def wait_load(t):
        pltpu.make_async_copy(x_hbm.at[pl.ds(0, _B)], xbs[t], lxs[t]).wait()
        pltpu.make_async_copy(idx_hbm.at[pl.ds(0, _B)], ibs[t], lis[t]).wait()

    def start_store(g, t):
        b = base0 + g * _B
        pltpu.async_copy(xbs[t], out_hbm.at[pl.ds(b, _B)], sts[t])

    def wait_store(t):
        pltpu.make_async_copy(xbs[t], out_hbm.at[pl.ds(0, _B)], sts[t]).wait()

    def compute(t):
        xb = xbs[t]

        def group(k, carry):
            iv16 = ibs[t][pl.ds(16 * k, 16)]
            for l in range(16):
                ds = iv16[l]
                row = 16 * k + l

                @plsc.parallel_loop(0, _NP, unroll=12)
                def _(j):
                    u = pe_v[ds, pl.ds(16 * j, 16)]
                    lo = jax.lax.bitcast_convert_type(
                        jnp.left_shift(u, 16), jnp.float32)
                    hi = jax.lax.bitcast_convert_type(
                        jnp.bitwise_and(u, jnp.int32(-65536)), jnp.float32)
                    plsc.addupdate(xb.at[row, pl.ds(32 * j, 16)], lo)
                    plsc.addupdate(xb.at[row, pl.ds(32 * j + 16, 16)], hi)
            return carry

        lax.fori_loop(0, _B // 16, group, 0)

    for t in range(3):
        start_load(t, t)
    for t in range(_NSLOT):
        pltpu.async_copy(xbs[t], dump_hbm.at[t], sts[t])

    def step(g, t):
        wait_load(t)
        compute(t)
        start_store(g, t)
        nt = (t + 3) % _NSLOT
        wait_store(nt)
        start_load(g + 3, nt)

    def body(h, carry):
        g0 = 4 * h
        for t in range(_NSLOT):
            step(g0 + t, t)
        return carry

    lax.fori_loop(0, chunks // 4, body, 0)

    for t in range(_NSLOT):
        wait_store(t)
    for t in range(3):
        wait_load(t)


def _pack_pe(pe2):
    pr = pe2.reshape(pe2.shape[0], _NP, 2, 16)
    lo = jax.lax.bitcast_convert_type(
        pr[:, :, 0, :].astype(jnp.bfloat16), jnp.uint16).astype(jnp.uint32)
    hi = jax.lax.bitcast_convert_type(
        pr[:, :, 1, :].astype(jnp.bfloat16), jnp.uint16).astype(jnp.uint32)
    packed = jnp.bitwise_or(lo, jnp.left_shift(hi, 16))
    return jax.lax.bitcast_convert_type(
        packed, jnp.int32).reshape(pe2.shape[0], _D // 2)


def kernel(x, layer_index, pe):
    n = x.shape[0]
    pe_p = _pack_pe(pe.reshape(pe.shape[0], _D))
    k = pl.kernel(
        _sc_body,
        out_type=(
            jax.ShapeDtypeStruct((n, _D), jnp.float32),
            jax.ShapeDtypeStruct((_NSLOT, _B, _D), jnp.float32),
        ),
        mesh=plsc.VectorSubcoreMesh(core_axis_name="c", subcore_axis_name="s",
                                    num_cores=_NC, num_subcores=_NS),
        scratch_types=(
            [pltpu.VMEM((100, _D // 2), jnp.int32)]
            + [pltpu.VMEM((_B, _D), jnp.float32) for _ in range(_NSLOT)]
            + [pltpu.VMEM((_B,), jnp.int32) for _ in range(_NSLOT)]
            + [pltpu.SemaphoreType.DMA for _ in range(3 * _NSLOT)]
        ),
    )
    out, _ = k(x, layer_index, pe_p)
    return out


# speedup vs baseline: 1.4956x; 1.4956x over previous
import jax
import jax.numpy as jnp
from jax import lax
from jax.experimental import pallas as pl
from jax.experimental.pallas import tpu as pltpu, tpu_sc as plsc

_D = 768
_B = 16
_NSLOT = 4
_NC, _NS = 2, 16
_NW = _NC * _NS
_NP = _D // 32
_SKEW = 4


def _sc_body(x_hbm, idx_hbm, pe_hbm, out_hbm, dump_hbm, pe_v, *rest):
    xbs = rest[0:4]
    ibs = rest[4:8]
    lxs = rest[8:12]
    lis = rest[12:16]
    sts = rest[16:20]

    c = lax.axis_index("c")
    s = lax.axis_index("s")
    wid = s * _NC + c
    rows_per_w = x_hbm.shape[0] // _NW
    chunks = rows_per_w // _B
    base0 = wid * rows_per_w

    pltpu.sync_copy(pe_hbm, pe_v)

    def start_load(g, t):
        b = base0 + jnp.minimum(g, chunks - 1) * _B
        pltpu.async_copy(x_hbm.at[pl.ds(b, _B)], xbs[t], lxs[t])
        pltpu.async_copy(idx_hbm.at[pl.ds(b, _B)], ibs[t], lis[t])

    def wait_load(t):
        pltpu.make_async_copy(x_hbm.at[pl.ds(0, _B)], xbs[t], lxs[t]).wait()
        pltpu.make_async_copy(idx_hbm.at[pl.ds(0, _B)], ibs[t], lis[t]).wait()

    def start_store(g, t):
        b = base0 + g * _B
        pltpu.async_copy(xbs[t], out_hbm.at[pl.ds(b, _B)], sts[t])

    def wait_store(t):
        pltpu.make_async_copy(xbs[t], out_hbm.at[pl.ds(0, _B)], sts[t]).wait()

    def compute(t):
        xb = xbs[t]

        def group(k, carry):
            iv16 = ibs[t][pl.ds(16 * k, 16)]
            for l in range(16):
                ds = iv16[l]
                row = 16 * k + l

                @plsc.parallel_loop(0, _NP, unroll=6)
                def _(j):
                    u = pe_v[ds, pl.ds(16 * j, 16)]
                    lo = jax.lax.bitcast_convert_type(
                        jnp.left_shift(u, 16), jnp.float32)
                    hi = jax.lax.bitcast_convert_type(
                        jnp.bitwise_and(u, jnp.int32(-65536)), jnp.float32)
                    plsc.addupdate(xb.at[row, pl.ds(32 * j, 16)], lo)
                    plsc.addupdate(xb.at[row, pl.ds(32 * j + 16, 16)], hi)
            return carry

        lax.fori_loop(0, _B // 16, group, 0)

    for t in range(3):
        start_load(t, t)
    for t in range(_NSLOT):
        pltpu.async_copy(xbs[t], dump_hbm.at[t], sts[t])

    def step(g, t):
        wait_load(t)
        compute(t)
        start_store(g, t)
        nt = (t + 3) % _NSLOT
        wait_store(nt)
        start_load(g + 3, nt)

    def body(h, carry):
        g0 = 4 * h
        for t in range(_NSLOT):
            step(g0 + t, t)
        return carry

    lax.fori_loop(0, chunks // 4, body, 0)

    for t in range(_NSLOT):
        wait_store(t)
    for t in range(3):
        wait_load(t)


def _pack_pe(pe2):
    pr = pe2.reshape(pe2.shape[0], _NP, 2, 16)
    lo = jax.lax.bitcast_convert_type(
        pr[:, :, 0, :].astype(jnp.bfloat16), jnp.uint16).astype(jnp.uint32)
    hi = jax.lax.bitcast_convert_type(
        pr[:, :, 1, :].astype(jnp.bfloat16), jnp.uint16).astype(jnp.uint32)
    packed = jnp.bitwise_or(lo, jnp.left_shift(hi, 16))
    return jax.lax.bitcast_convert_type(
        packed, jnp.int32).reshape(pe2.shape[0], _D // 2)


def kernel(x, layer_index, pe):
    n = x.shape[0]
    pe_p = _pack_pe(pe.reshape(pe.shape[0], _D))
    k = pl.kernel(
        _sc_body,
        out_type=(
            jax.ShapeDtypeStruct((n, _D), jnp.float32),
            jax.ShapeDtypeStruct((_NSLOT, _B, _D), jnp.float32),
        ),
        mesh=plsc.VectorSubcoreMesh(core_axis_name="c", subcore_axis_name="s",
                                    num_cores=_NC, num_subcores=_NS),
        scratch_types=(
            [pltpu.VMEM((100, _D // 2), jnp.int32)]
            + [pltpu.VMEM((_B, _D), jnp.float32) for _ in range(_NSLOT)]
            + [pltpu.VMEM((_B,), jnp.int32) for _ in range(_NSLOT)]
            + [pltpu.SemaphoreType.DMA for _ in range(3 * _NSLOT)]
        ),
    )
    out, _ = k(x, layer_index, pe_p)
    return out
